# baseline (speedup 1.0000x reference)
import functools

import jax
import jax.numpy as jnp
from jax import lax
from jax.experimental import pallas as pl
from jax.experimental.pallas import tpu as pltpu
from jax.experimental.pallas import tpu_sc as plsc

B = 16384
E = 64
V = 100000
NC = 2
NS = 16
NW = NC * NS
CHUNK = B // NW
LANES = 16
G = 128
NCHUNK = CHUNK // G
CROWS = CHUNK // LANES

_mesh = plsc.VectorSubcoreMesh(
    core_axis_name="c", subcore_axis_name="s", num_cores=NC, num_subcores=NS
)


_TBLK = 10240
V2S = 51200
_NBLK2 = pl.cdiv(V, _TBLK) - 1


def _transpose_body(i1_ref, i2_ref, o_ref):
    eye = jnp.eye(E, dtype=jnp.float32)
    dn = (((0,), (0,)), ((), ()))
    o_ref[:, 0:E] = jax.lax.dot_general(
        i1_ref[...], eye, dn, preferred_element_type=jnp.float32
    )
    o_ref[:, E:2 * E] = jax.lax.dot_general(
        i2_ref[...], eye, dn, preferred_element_type=jnp.float32
    )


_transpose = pl.pallas_call(
    _transpose_body,
    grid=(V2S // _TBLK,),
    in_specs=[
        pl.BlockSpec((E, _TBLK), lambda i: (0, i)),
        pl.BlockSpec((E, _TBLK), lambda i: (0, jnp.minimum(i + V2S // _TBLK, _NBLK2))),
    ],
    out_specs=pl.BlockSpec((_TBLK, 2 * E), lambda i: (i, 0)),
    out_shape=jax.ShapeDtypeStruct((V2S, 2 * E), jnp.float32),
)


@functools.partial(
    pl.kernel,
    mesh=_mesh,
    compiler_params=pltpu.CompilerParams(use_tc_tiling_on_sc=True),
    out_type=(
        jax.ShapeDtypeStruct((B, E), jnp.float32),
        jax.ShapeDtypeStruct((NW, CROWS, LANES), jnp.float32),
    ),
    scratch_types=[
        pltpu.VMEM((G, 2 * E), jnp.float32),
        pltpu.VMEM((G, 2 * E), jnp.float32),
        pltpu.VMEM((G, E), jnp.float32),
        pltpu.VMEM((G, E), jnp.float32),
        pltpu.VMEM((CROWS, LANES), jnp.float32),
        pltpu.VMEM((CHUNK,), jnp.int32),
        pltpu.VMEM((CHUNK,), jnp.int32),
        pltpu.VMEM((NCHUNK, G), jnp.int32),
        pltpu.VMEM((CHUNK,), jnp.float32),
        pltpu.VMEM((CHUNK,), jnp.float32),
        pltpu.SemaphoreType.DMA,
        pltpu.SemaphoreType.DMA,
        pltpu.SemaphoreType.DMA,
        pltpu.SemaphoreType.DMA,
        pltpu.SemaphoreType.DMA,
    ],
)
def _stage1a(
    uid_hbm, mid_hbm, ue_hbm, ub_hbm, mb_hbm,
    urows_hbm, bsum_hbm,
    u0_v, u1_v, c0_v, c1_v, bsum_v, uidx_v, midx_v, urow_v, ub_v, mb_v,
    sem_u0, sem_u1, sem_ub, sem_mb, sem_w,
):
    wid = lax.axis_index("s") * NC + lax.axis_index("c")
    base = wid * CHUNK

    ubuf = (u0_v, u1_v)
    cbuf = (c0_v, c1_v)
    usem = (sem_u0, sem_u1)

    pltpu.sync_copy(uid_hbm.at[pl.ds(base, CHUNK)], uidx_v)
    pltpu.sync_copy(mid_hbm.at[pl.ds(base, CHUNK)], midx_v)

    cp_ub = pltpu.async_copy(ub_hbm.at[uidx_v], ub_v, sem_ub)
    cp_mb = pltpu.async_copy(mb_hbm.at[midx_v], mb_v, sem_mb)

    for h in range(NCHUNK):
        def xform(k, carry, h=h):
            sl = pl.ds(k * LANES, LANES)
            v = uidx_v[pl.ds(h * G + k * LANES, LANES)]
            urow_v[h, sl] = jnp.where(v >= V2S, v - V2S, v)
            return carry

        lax.fori_loop(0, G // LANES, xform, 0)

    def enqueue_chunk(h, p):
        pltpu.async_copy(ue_hbm.at[urow_v.at[h]], ubuf[p], usem[p])

    def drain_chunk(p):
        pltpu.make_async_copy(ue_hbm.at[pl.ds(0, G), :], ubuf[p], usem[p]).wait()

    def compact_chunk(h, p):
        u = ubuf[p]
        c = cbuf[p]

        def grp_body(k, carry):
            uvec = uidx_v[pl.ds(h * G + k * LANES, LANES)]
            for j in range(LANES):
                uoff = jnp.where(uvec[j] >= V2S, E, 0)
                i = k * LANES + j
                for q in range(E // LANES):
                    c[i, pl.ds(q * LANES, LANES)] = u[i, pl.ds(uoff + q * LANES, LANES)]
            return carry

        lax.fori_loop(0, G // LANES, grp_body, 0)
        pltpu.async_copy(c, urows_hbm.at[pl.ds(base + h * G, G), :], sem_w)

    enqueue_chunk(0, 0)
    for h in range(NCHUNK):
        p = h % 2
        if h + 1 < NCHUNK:
            enqueue_chunk(h + 1, 1 - p)
        drain_chunk(p)
        if h >= 2:
            pltpu.make_async_copy(
                urows_hbm.at[pl.ds(0, G), :], cbuf[p], sem_w
            ).wait()
        compact_chunk(h, p)
    for _ in range(2):
        pltpu.make_async_copy(urows_hbm.at[pl.ds(0, G), :], c0_v, sem_w).wait()

    cp_ub.wait()
    cp_mb.wait()
    for k in range(CROWS):
        sl = pl.ds(k * LANES, LANES)
        bsum_v[k, :] = ub_v[sl] + mb_v[sl]
    pltpu.sync_copy(bsum_v, bsum_hbm.at[wid])


@functools.partial(
    pl.kernel,
    mesh=_mesh,
    compiler_params=pltpu.CompilerParams(use_tc_tiling_on_sc=True),
    out_type=jax.ShapeDtypeStruct((NW, LANES), jnp.float32),
    scratch_types=[
        pltpu.VMEM((G, 2 * E), jnp.float32),
        pltpu.VMEM((G, 2 * E), jnp.float32),
        pltpu.VMEM((G, E), jnp.float32),
        pltpu.VMEM((G, E), jnp.float32),
        pltpu.VMEM((LANES,), jnp.float32),
        pltpu.VMEM((CHUNK,), jnp.int32),
        pltpu.VMEM((NCHUNK, G), jnp.int32),
        pltpu.SemaphoreType.DMA,
        pltpu.SemaphoreType.DMA,
        pltpu.SemaphoreType.DMA,
        pltpu.SemaphoreType.DMA,
    ],
)
def _stage1b(
    mid_hbm, me_hbm, urows_hbm,
    partials_hbm,
    m0_v, m1_v, r0_v, r1_v, acc_v, midx_v, mrow_v,
    sem_m0, sem_m1, sem_r0, sem_r1,
):
    wid = lax.axis_index("s") * NC + lax.axis_index("c")
    base = wid * CHUNK

    mbuf = (m0_v, m1_v)
    rbuf = (r0_v, r1_v)
    msem = (sem_m0, sem_m1)
    rsem = (sem_r0, sem_r1)

    pltpu.sync_copy(mid_hbm.at[pl.ds(base, CHUNK)], midx_v)

    for h in range(NCHUNK):
        def xform(k, carry, h=h):
            sl = pl.ds(k * LANES, LANES)
            w = midx_v[pl.ds(h * G + k * LANES, LANES)]
            mrow_v[h, sl] = jnp.where(w >= V2S, w - V2S, w)
            return carry

        lax.fori_loop(0, G // LANES, xform, 0)

    def enqueue_chunk(h, p):
        pltpu.async_copy(me_hbm.at[mrow_v.at[h]], mbuf[p], msem[p])
        pltpu.async_copy(
            urows_hbm.at[pl.ds(base + h * G, G), :], rbuf[p], rsem[p]
        )

    def drain_chunk(p):
        pltpu.make_async_copy(me_hbm.at[pl.ds(0, G), :], mbuf[p], msem[p]).wait()
        pltpu.make_async_copy(
            urows_hbm.at[pl.ds(0, G), :], rbuf[p], rsem[p]
        ).wait()

    def compute_chunk(h, p, acc):
        m = mbuf[p]
        r = rbuf[p]

        def grp_body(k, a):
            mvec = midx_v[pl.ds(h * G + k * LANES, LANES)]
            for j in range(LANES):
                moff = jnp.where(mvec[j] >= V2S, E, 0)
                i = k * LANES + j
                t = r[i, pl.ds(0, LANES)] * m[i, pl.ds(moff, LANES)]
                for q in range(1, E // LANES):
                    t += (
                        r[i, pl.ds(q * LANES, LANES)]
                        * m[i, pl.ds(moff + q * LANES, LANES)]
                    )
                a = a + t
            return a

        return lax.fori_loop(0, G // LANES, grp_body, acc)

    acc = jnp.zeros((LANES,), jnp.float32)
    enqueue_chunk(0, 0)
    for h in range(NCHUNK):
        p = h % 2
        if h + 1 < NCHUNK:
            enqueue_chunk(h + 1, 1 - p)
        drain_chunk(p)
        acc = compute_chunk(h, p, acc)

    acc_v[...] = acc
    pltpu.sync_copy(acc_v, partials_hbm.at[wid])


def _stage2_body(p_ref, b_ref, o_ref):
    s = jnp.sum(p_ref[...])
    o_ref[...] = jax.nn.sigmoid(s + b_ref[...])


_stage2 = pl.pallas_call(
    _stage2_body,
    out_shape=jax.ShapeDtypeStruct((B // 128, 128), jnp.float32),
)


def kernel(inputs, user_embedding, user_bias, movie_embedding, movie_bias):
    uid = inputs[:, 0].astype(jnp.int32)
    mid = inputs[:, 1].astype(jnp.int32)
    ut = user_embedding.T
    mt = movie_embedding.T
    ub_t = user_bias[:, 0]
    mb_t = movie_bias[:, 0]
    ue_rm = _transpose(ut, ut)
    urows, bsum = _stage1a(uid, mid, ue_rm, ub_t, mb_t)
    me_rm = _transpose(mt, mt)
    partials = _stage1b(mid, me_rm, urows)
    out = _stage2(partials, bsum.reshape(B // 128, 128))
    return out.reshape(B, 1)


# speedup vs baseline: 1.0132x; 1.0132x over previous
import functools

import jax
import jax.numpy as jnp
from jax import lax
from jax.experimental import pallas as pl
from jax.experimental.pallas import tpu as pltpu
from jax.experimental.pallas import tpu_sc as plsc

B = 16384
E = 64
V = 100000
NC = 2
NS = 16
NW = NC * NS
CHUNK = B // NW
LANES = 16
G = 128
NCHUNK = CHUNK // G
CROWS = CHUNK // LANES

_mesh = plsc.VectorSubcoreMesh(
    core_axis_name="c", subcore_axis_name="s", num_cores=NC, num_subcores=NS
)


_TBLK = 10240
V2S = 51200
_NBLK2 = pl.cdiv(V, _TBLK) - 1


def _transpose_body(i1_ref, i2_ref, o_ref):
    eye = jnp.eye(E, dtype=jnp.float32)
    dn = (((0,), (0,)), ((), ()))
    o_ref[:, 0:E] = jax.lax.dot_general(
        i1_ref[...], eye, dn, preferred_element_type=jnp.float32
    )
    o_ref[:, E:2 * E] = jax.lax.dot_general(
        i2_ref[...], eye, dn, preferred_element_type=jnp.float32
    )


_transpose = pl.pallas_call(
    _transpose_body,
    grid=(V2S // _TBLK,),
    in_specs=[
        pl.BlockSpec((E, _TBLK), lambda i: (0, i)),
        pl.BlockSpec((E, _TBLK), lambda i: (0, jnp.minimum(i + V2S // _TBLK, _NBLK2))),
    ],
    out_specs=pl.BlockSpec((_TBLK, 2 * E), lambda i: (i, 0)),
    out_shape=jax.ShapeDtypeStruct((V2S, 2 * E), jnp.float32),
)


@functools.partial(
    pl.kernel,
    mesh=_mesh,
    compiler_params=pltpu.CompilerParams(use_tc_tiling_on_sc=True),
    out_type=(
        jax.ShapeDtypeStruct((NW, LANES), jnp.float32),
        jax.ShapeDtypeStruct((NW, CROWS, LANES), jnp.float32),
    ),
    scratch_types=[
        pltpu.VMEM((G, 2 * E), jnp.float32),
        pltpu.VMEM((G, 2 * E), jnp.float32),
        pltpu.VMEM((G, 2 * E), jnp.float32),
        pltpu.VMEM((G, 2 * E), jnp.float32),
        pltpu.VMEM((CROWS, LANES), jnp.float32),
        pltpu.VMEM((LANES,), jnp.float32),
        pltpu.VMEM((CHUNK,), jnp.int32),
        pltpu.VMEM((CHUNK,), jnp.int32),
        pltpu.VMEM((NCHUNK, G), jnp.int32),
        pltpu.VMEM((NCHUNK, G), jnp.int32),
        pltpu.VMEM((CHUNK,), jnp.float32),
        pltpu.VMEM((CHUNK,), jnp.float32),
        pltpu.SemaphoreType.DMA,
        pltpu.SemaphoreType.DMA,
        pltpu.SemaphoreType.DMA,
        pltpu.SemaphoreType.DMA,
        pltpu.SemaphoreType.DMA,
        pltpu.SemaphoreType.DMA,
    ],
)
def _stage1(
    uid_hbm, mid_hbm, ue_hbm, me_hbm, ub_hbm, mb_hbm,
    partials_hbm, bsum_hbm,
    u0_v, u1_v, m0_v, m1_v, bsum_v, acc_v, uidx_v, midx_v, urow_v, mrow_v,
    ub_v, mb_v,
    sem_u0, sem_u1, sem_m0, sem_m1, sem_ub, sem_mb,
):
    wid = lax.axis_index("s") * NC + lax.axis_index("c")
    base = wid * CHUNK

    ubuf = (u0_v, u1_v)
    mbuf = (m0_v, m1_v)
    usem = (sem_u0, sem_u1)
    msem = (sem_m0, sem_m1)

    pltpu.sync_copy(uid_hbm.at[pl.ds(base, CHUNK)], uidx_v)
    pltpu.sync_copy(mid_hbm.at[pl.ds(base, CHUNK)], midx_v)

    cp_ub = pltpu.async_copy(ub_hbm.at[uidx_v], ub_v, sem_ub)
    cp_mb = pltpu.async_copy(mb_hbm.at[midx_v], mb_v, sem_mb)

    for h in range(NCHUNK):
        def xform(k, carry, h=h):
            sl = pl.ds(k * LANES, LANES)
            v = uidx_v[pl.ds(h * G + k * LANES, LANES)]
            urow_v[h, sl] = jnp.where(v >= V2S, v - V2S, v)
            w = midx_v[pl.ds(h * G + k * LANES, LANES)]
            mrow_v[h, sl] = jnp.where(w >= V2S, w - V2S, w)
            return carry

        lax.fori_loop(0, G // LANES, xform, 0)

    def enqueue_chunk(h, p):
        pltpu.async_copy(ue_hbm.at[urow_v.at[h]], ubuf[p], usem[p])
        pltpu.async_copy(me_hbm.at[mrow_v.at[h]], mbuf[p], msem[p])

    def drain_chunk(p):
        pltpu.make_async_copy(ue_hbm.at[pl.ds(0, G), :], ubuf[p], usem[p]).wait()
        pltpu.make_async_copy(me_hbm.at[pl.ds(0, G), :], mbuf[p], msem[p]).wait()

    def compute_chunk(h, p, acc):
        u = ubuf[p]
        m = mbuf[p]

        def grp_body(k, a):
            uvec = uidx_v[pl.ds(h * G + k * LANES, LANES)]
            mvec = midx_v[pl.ds(h * G + k * LANES, LANES)]
            for j in range(LANES):
                uoff = jnp.where(uvec[j] >= V2S, E, 0)
                moff = jnp.where(mvec[j] >= V2S, E, 0)
                i = k * LANES + j
                t = u[i, pl.ds(uoff, LANES)] * m[i, pl.ds(moff, LANES)]
                for q in range(1, E // LANES):
                    t += (
                        u[i, pl.ds(uoff + q * LANES, LANES)]
                        * m[i, pl.ds(moff + q * LANES, LANES)]
                    )
                a = a + t
            return a

        return lax.fori_loop(0, G // LANES, grp_body, acc)

    acc = jnp.zeros((LANES,), jnp.float32)
    enqueue_chunk(0, 0)
    for h in range(NCHUNK):
        p = h % 2
        if h + 1 < NCHUNK:
            enqueue_chunk(h + 1, 1 - p)
        drain_chunk(p)
        acc = compute_chunk(h, p, acc)

    acc_v[...] = acc
    pltpu.sync_copy(acc_v, partials_hbm.at[wid])

    cp_ub.wait()
    cp_mb.wait()
    for k in range(CROWS):
        sl = pl.ds(k * LANES, LANES)
        bsum_v[k, :] = ub_v[sl] + mb_v[sl]
    pltpu.sync_copy(bsum_v, bsum_hbm.at[wid])


def _stage2_body(p_ref, b_ref, o_ref):
    s = jnp.sum(p_ref[...])
    o_ref[...] = jax.nn.sigmoid(s + b_ref[...])


_stage2 = pl.pallas_call(
    _stage2_body,
    out_shape=jax.ShapeDtypeStruct((B // 128, 128), jnp.float32),
)


def kernel(inputs, user_embedding, user_bias, movie_embedding, movie_bias):
    uid = inputs[:, 0].astype(jnp.int32)
    mid = inputs[:, 1].astype(jnp.int32)
    ut = user_embedding.T
    mt = movie_embedding.T
    ue_rm = _transpose(ut, ut)
    me_rm = _transpose(mt, mt)
    ub_t = user_bias[:, 0]
    mb_t = movie_bias[:, 0]
    partials, bsum = _stage1(uid, mid, ue_rm, me_rm, ub_t, mb_t)
    out = _stage2(partials, bsum.reshape(B // 128, 128))
    return out.reshape(B, 1)
